# initial kernel scaffold (unmeasured)
import jax
import jax.numpy as jnp
from jax import lax
from jax.experimental import pallas as pl
from jax.experimental.pallas import tpu as pltpu

N_DEV = 8
M_CHUNK = 512


def kernel(x, w_mat, scale_x, scale_w):
    m_total, k_per = x.shape
    _, n = w_mat.shape

    def body(x_ref, w_ref, sx_ref, sw_ref, out_ref, comm_ref,
             send_sems, recv_sems):
        d = lax.axis_index("i")
        left = lax.rem(d - 1 + N_DEV, N_DEV)
        right = lax.rem(d + 1, N_DEV)

        barrier_sem = pltpu.get_barrier_semaphore()
        for nbr in (left, right):
            pl.semaphore_signal(
                barrier_sem, inc=1,
                device_id=(nbr,), device_id_type=pl.DeviceIdType.MESH,
            )
        pl.semaphore_wait(barrier_sem, 2)

        def partial(c):
            xc = x_ref[pl.ds(c * M_CHUNK, M_CHUNK), :]
            return lax.dot_general(
                xc, w_ref[:, :],
                (((1,), (0,)), ((), ())),
                preferred_element_type=jnp.int32,
            )

        c0 = lax.rem(d - 1 + N_DEV, N_DEV)
        comm_ref[0, :, :] = partial(c0)

        for s in range(N_DEV - 1):
            send_slot = s % 2
            recv_slot = (s + 1) % 2
            rdma = pltpu.make_async_remote_copy(
                src_ref=comm_ref.at[send_slot],
                dst_ref=comm_ref.at[recv_slot],
                send_sem=send_sems.at[send_slot],
                recv_sem=recv_sems.at[s],
                device_id=(right,),
                device_id_type=pl.DeviceIdType.MESH,
            )
            rdma.start()
            rdma.wait()
            c = lax.rem(d - 2 - s + 2 * N_DEV, N_DEV)
            comm_ref[recv_slot, :, :] = comm_ref[recv_slot, :, :] + partial(c)

        acc = comm_ref[(N_DEV - 1) % 2, :, :]
        y = acc.astype(jnp.float32) * (sx_ref[0] * sw_ref[0])
        yc = jnp.clip(y, -60.0, 60.0)
        out_ref[:, :] = y / (1.0 + jnp.exp(-yc))

    return pl.pallas_call(
        body,
        out_shape=jax.ShapeDtypeStruct((M_CHUNK, n), jnp.float32),
        in_specs=[
            pl.BlockSpec(memory_space=pltpu.VMEM),
            pl.BlockSpec(memory_space=pltpu.VMEM),
            pl.BlockSpec(memory_space=pltpu.SMEM),
            pl.BlockSpec(memory_space=pltpu.SMEM),
        ],
        out_specs=pl.BlockSpec(memory_space=pltpu.VMEM),
        scratch_shapes=[
            pltpu.VMEM((2, M_CHUNK, n), jnp.int32),
            pltpu.SemaphoreType.DMA((2,)),
            pltpu.SemaphoreType.DMA((N_DEV - 1,)),
        ],
        compiler_params=pltpu.CompilerParams(collective_id=0),
    )(x, w_mat, scale_x, scale_w)


# baseline (device time: 1341469 ns/iter reference)
import jax
import jax.numpy as jnp
from jax import lax
from jax.experimental import pallas as pl
from jax.experimental.pallas import tpu as pltpu

N_DEV = 8
M_CHUNK = 512


def kernel(x, w_mat, scale_x, scale_w):
    m_total, k_per = x.shape
    _, n = w_mat.shape

    def body(x_ref, w_ref, sx_ref, sw_ref, out_ref, comm_ref,
             send_sems, recv_sems):
        d = lax.axis_index("i")
        left = lax.rem(d - 1 + N_DEV, N_DEV)
        right = lax.rem(d + 1, N_DEV)

        barrier_sem = pltpu.get_barrier_semaphore()
        for nbr in (left, right):
            pl.semaphore_signal(
                barrier_sem, inc=1,
                device_id=(nbr,), device_id_type=pl.DeviceIdType.MESH,
            )
        pl.semaphore_wait(barrier_sem, 2)

        def partial(c):
            xc = x_ref[pl.ds(c * M_CHUNK, M_CHUNK), :]
            return lax.dot_general(
                xc, w_ref[:, :],
                (((1,), (0,)), ((), ())),
                preferred_element_type=jnp.int32,
            )

        c0 = lax.rem(d - 1 + N_DEV, N_DEV)
        comm_ref[0, :, :] = partial(c0)

        for s in range(N_DEV - 1):
            send_slot = s % 2
            recv_slot = (s + 1) % 2
            rdma = pltpu.make_async_remote_copy(
                src_ref=comm_ref.at[send_slot],
                dst_ref=comm_ref.at[recv_slot],
                send_sem=send_sems.at[send_slot],
                recv_sem=recv_sems.at[s],
                device_id=(right,),
                device_id_type=pl.DeviceIdType.MESH,
            )
            rdma.start()
            rdma.wait()
            c = lax.rem(d - 2 - s + 2 * N_DEV, N_DEV)
            comm_ref[recv_slot, :, :] = comm_ref[recv_slot, :, :] + partial(c)

        acc = comm_ref[(N_DEV - 1) % 2, :, :]
        y = acc.astype(jnp.float32) * (sx_ref[0] * sw_ref[0])
        yc = jnp.clip(y, -60.0, 60.0)
        out_ref[:, :] = y / (1.0 + jnp.exp(-yc))

    return pl.pallas_call(
        body,
        out_shape=jax.ShapeDtypeStruct((M_CHUNK, n), jnp.float32),
        in_specs=[
            pl.BlockSpec(memory_space=pltpu.VMEM),
            pl.BlockSpec(memory_space=pltpu.VMEM),
            pl.BlockSpec(memory_space=pltpu.SMEM),
            pl.BlockSpec(memory_space=pltpu.SMEM),
        ],
        out_specs=pl.BlockSpec(memory_space=pltpu.VMEM),
        scratch_shapes=[
            pltpu.VMEM((2, M_CHUNK, n), jnp.int32),
            pltpu.SemaphoreType.DMA((2,)),
            pltpu.SemaphoreType.DMA((N_DEV - 1,)),
        ],
        compiler_params=pltpu.CompilerParams(
            collective_id=0,
            vmem_limit_bytes=128 * 1024 * 1024,
        ),
    )(x, w_mat, scale_x, scale_w)


# device time: 715157 ns/iter; 1.8758x vs baseline; 1.8758x over previous
import jax
import jax.numpy as jnp
from jax import lax
from jax.experimental import pallas as pl
from jax.experimental.pallas import tpu as pltpu

N_DEV = 8
M_CHUNK = 512


def kernel(x, w_mat, scale_x, scale_w):
    m_total, k_per = x.shape
    _, n = w_mat.shape
    half = n // 2

    def body(x_ref, w_ref, sx_ref, sw_ref, out_ref, comm_r, comm_l,
             send_sems_r, send_sems_l, recv_sems_r, recv_sems_l):
        d = lax.axis_index("i")
        left = lax.rem(d - 1 + N_DEV, N_DEV)
        right = lax.rem(d + 1, N_DEV)

        barrier_sem = pltpu.get_barrier_semaphore()
        for nbr in (left, right):
            pl.semaphore_signal(
                barrier_sem, inc=1,
                device_id=(nbr,), device_id_type=pl.DeviceIdType.MESH,
            )
        pl.semaphore_wait(barrier_sem, 2)

        def partial(c, lo):
            xc = x_ref[pl.ds(c * M_CHUNK, M_CHUNK), :]
            wc = w_ref[:, lo:lo + half]
            return lax.dot_general(
                xc, wc,
                (((1,), (0,)), ((), ())),
                preferred_element_type=jnp.int32,
            )

        comm_r[0, :, :] = partial(lax.rem(d - 1 + N_DEV, N_DEV), 0)
        comm_l[0, :, :] = partial(lax.rem(d + 1, N_DEV), half)

        for s in range(N_DEV - 1):
            ss = s % 2
            rs = (s + 1) % 2
            rdma_r = pltpu.make_async_remote_copy(
                src_ref=comm_r.at[ss],
                dst_ref=comm_r.at[rs],
                send_sem=send_sems_r.at[ss],
                recv_sem=recv_sems_r.at[s],
                device_id=(right,),
                device_id_type=pl.DeviceIdType.MESH,
            )
            rdma_l = pltpu.make_async_remote_copy(
                src_ref=comm_l.at[ss],
                dst_ref=comm_l.at[rs],
                send_sem=send_sems_l.at[ss],
                recv_sem=recv_sems_l.at[s],
                device_id=(left,),
                device_id_type=pl.DeviceIdType.MESH,
            )
            rdma_r.start()
            rdma_l.start()
            rdma_r.wait()
            rdma_l.wait()
            c_r = lax.rem(d - 2 - s + 2 * N_DEV, N_DEV)
            c_l = lax.rem(d + 2 + s, N_DEV)
            comm_r[rs, :, :] = comm_r[rs, :, :] + partial(c_r, 0)
            comm_l[rs, :, :] = comm_l[rs, :, :] + partial(c_l, half)

        scale = sx_ref[0] * sw_ref[0]
        y_r = comm_r[(N_DEV - 1) % 2, :, :].astype(jnp.float32) * scale
        y_l = comm_l[(N_DEV - 1) % 2, :, :].astype(jnp.float32) * scale
        out_ref[:, 0:half] = y_r / (1.0 + jnp.exp(-jnp.clip(y_r, -60.0, 60.0)))
        out_ref[:, half:n] = y_l / (1.0 + jnp.exp(-jnp.clip(y_l, -60.0, 60.0)))

    return pl.pallas_call(
        body,
        out_shape=jax.ShapeDtypeStruct((M_CHUNK, n), jnp.float32),
        in_specs=[
            pl.BlockSpec(memory_space=pltpu.VMEM),
            pl.BlockSpec(memory_space=pltpu.VMEM),
            pl.BlockSpec(memory_space=pltpu.SMEM),
            pl.BlockSpec(memory_space=pltpu.SMEM),
        ],
        out_specs=pl.BlockSpec(memory_space=pltpu.VMEM),
        scratch_shapes=[
            pltpu.VMEM((2, M_CHUNK, half), jnp.int32),
            pltpu.VMEM((2, M_CHUNK, half), jnp.int32),
            pltpu.SemaphoreType.DMA((2,)),
            pltpu.SemaphoreType.DMA((2,)),
            pltpu.SemaphoreType.DMA((N_DEV - 1,)),
            pltpu.SemaphoreType.DMA((N_DEV - 1,)),
        ],
        compiler_params=pltpu.CompilerParams(
            collective_id=0,
            vmem_limit_bytes=128 * 1024 * 1024,
        ),
    )(x, w_mat, scale_x, scale_w)


# device time: 692524 ns/iter; 1.9371x vs baseline; 1.0327x over previous
import jax
import jax.numpy as jnp
from jax import lax
from jax.experimental import pallas as pl
from jax.experimental.pallas import tpu as pltpu

N_DEV = 8
M_CHUNK = 512


def kernel(x, w_mat, scale_x, scale_w):
    m_total, k_per = x.shape
    _, n = w_mat.shape
    half = n // 2

    def body(x_ref, w_ref, sx_ref, sw_ref, out_ref, comm_r, comm_l,
             send_sems_r, send_sems_l, recv_sems_r, recv_sems_l):
        d = lax.axis_index("i")
        left = lax.rem(d - 1 + N_DEV, N_DEV)
        right = lax.rem(d + 1, N_DEV)

        barrier_sem = pltpu.get_barrier_semaphore()
        for nbr in (left, right):
            pl.semaphore_signal(
                barrier_sem, inc=1,
                device_id=(nbr,), device_id_type=pl.DeviceIdType.MESH,
            )
        pl.semaphore_wait(barrier_sem, 2)

        def partial(c, lo):
            xc = x_ref[pl.ds(c * M_CHUNK, M_CHUNK), :]
            wc = w_ref[:, lo:lo + half]
            return lax.dot_general(
                xc, wc,
                (((1,), (0,)), ((), ())),
                preferred_element_type=jnp.int32,
            )

        comm_r[0, :, :] = partial(lax.rem(d - 1 + N_DEV, N_DEV), 0)
        comm_l[0, :, :] = partial(lax.rem(d + 1, N_DEV), half)

        for s in range(N_DEV - 1):
            ss = s % 2
            rs = (s + 1) % 2
            rdma_r = pltpu.make_async_remote_copy(
                src_ref=comm_r.at[ss],
                dst_ref=comm_r.at[rs],
                send_sem=send_sems_r.at[ss],
                recv_sem=recv_sems_r.at[s],
                device_id=(right,),
                device_id_type=pl.DeviceIdType.MESH,
            )
            rdma_l = pltpu.make_async_remote_copy(
                src_ref=comm_l.at[ss],
                dst_ref=comm_l.at[rs],
                send_sem=send_sems_l.at[ss],
                recv_sem=recv_sems_l.at[s],
                device_id=(left,),
                device_id_type=pl.DeviceIdType.MESH,
            )
            rdma_r.start()
            rdma_l.start()
            c_r = lax.rem(d - 2 - s + 2 * N_DEV, N_DEV)
            c_l = lax.rem(d + 2 + s, N_DEV)
            out_ref[:, 0:half] = partial(c_r, 0).astype(jnp.float32)
            out_ref[:, half:n] = partial(c_l, half).astype(jnp.float32)
            rdma_r.wait()
            comm_r[rs, :, :] = comm_r[rs, :, :] + out_ref[:, 0:half].astype(
                jnp.int32)
            rdma_l.wait()
            comm_l[rs, :, :] = comm_l[rs, :, :] + out_ref[:, half:n].astype(
                jnp.int32)

        scale = sx_ref[0] * sw_ref[0]
        y_r = comm_r[(N_DEV - 1) % 2, :, :].astype(jnp.float32) * scale
        y_l = comm_l[(N_DEV - 1) % 2, :, :].astype(jnp.float32) * scale
        out_ref[:, 0:half] = y_r / (1.0 + jnp.exp(-jnp.clip(y_r, -60.0, 60.0)))
        out_ref[:, half:n] = y_l / (1.0 + jnp.exp(-jnp.clip(y_l, -60.0, 60.0)))

    return pl.pallas_call(
        body,
        out_shape=jax.ShapeDtypeStruct((M_CHUNK, n), jnp.float32),
        in_specs=[
            pl.BlockSpec(memory_space=pltpu.VMEM),
            pl.BlockSpec(memory_space=pltpu.VMEM),
            pl.BlockSpec(memory_space=pltpu.SMEM),
            pl.BlockSpec(memory_space=pltpu.SMEM),
        ],
        out_specs=pl.BlockSpec(memory_space=pltpu.VMEM),
        scratch_shapes=[
            pltpu.VMEM((2, M_CHUNK, half), jnp.int32),
            pltpu.VMEM((2, M_CHUNK, half), jnp.int32),
            pltpu.SemaphoreType.DMA((2,)),
            pltpu.SemaphoreType.DMA((2,)),
            pltpu.SemaphoreType.DMA((N_DEV - 1,)),
            pltpu.SemaphoreType.DMA((N_DEV - 1,)),
        ],
        compiler_params=pltpu.CompilerParams(
            collective_id=0,
            vmem_limit_bytes=128 * 1024 * 1024,
        ),
    )(x, w_mat, scale_x, scale_w)


# device time: 683686 ns/iter; 1.9621x vs baseline; 1.0129x over previous
import jax
import jax.numpy as jnp
from jax import lax
from jax.experimental import pallas as pl
from jax.experimental.pallas import tpu as pltpu

N_DEV = 8
M_CHUNK = 512


def kernel(x, w_mat, scale_x, scale_w):
    m_total, k_per = x.shape
    _, n = w_mat.shape
    half = n // 2

    def body(x_ref, w_ref, sx_ref, sw_ref, out_ref, comm_r, comm_l,
             send_sems_r, send_sems_l, recv_sems_r, recv_sems_l):
        d = lax.axis_index("i")
        left = lax.rem(d - 1 + N_DEV, N_DEV)
        right = lax.rem(d + 1, N_DEV)

        barrier_sem = pltpu.get_barrier_semaphore()
        for nbr in (left, right):
            pl.semaphore_signal(
                barrier_sem, inc=1,
                device_id=(nbr,), device_id_type=pl.DeviceIdType.MESH,
            )
        pl.semaphore_wait(barrier_sem, 2)

        def partial(c, lo):
            xc = x_ref[pl.ds(c * M_CHUNK, M_CHUNK), :]
            wc = w_ref[:, lo:lo + half]
            return lax.dot_general(
                xc, wc,
                (((1,), (0,)), ((), ())),
                preferred_element_type=jnp.int32,
            )

        comm_r[0, :, :] = partial(lax.rem(d - 1 + N_DEV, N_DEV), 0)
        comm_l[0, :, :] = partial(lax.rem(d + 1, N_DEV), half)

        scale = sx_ref[0] * sw_ref[0]
        sub = half // 2

        for s in range(N_DEV - 1):
            ss = s % 2
            rs = (s + 1) % 2
            descs = []
            for comm, ssems, rsems, tgt, base in (
                (comm_r, send_sems_r, recv_sems_r, right, 0),
                (comm_l, send_sems_l, recv_sems_l, left, half),
            ):
                for t in range(2):
                    cols = slice(t * sub, (t + 1) * sub)
                    rdma = pltpu.make_async_remote_copy(
                        src_ref=comm.at[ss, :, cols],
                        dst_ref=comm.at[rs, :, cols],
                        send_sem=ssems.at[ss, t],
                        recv_sem=rsems.at[s, t],
                        device_id=(tgt,),
                        device_id_type=pl.DeviceIdType.MESH,
                    )
                    rdma.start()
                    descs.append((rdma, comm, cols, base))
            c_r = lax.rem(d - 2 - s + 2 * N_DEV, N_DEV)
            c_l = lax.rem(d + 2 + s, N_DEV)
            out_ref[:, 0:half] = partial(c_r, 0).astype(jnp.float32)
            out_ref[:, half:n] = partial(c_l, half).astype(jnp.float32)
            last = s == N_DEV - 2
            for rdma, comm, cols, base in (
                descs[0], descs[2], descs[1], descs[3],
            ):
                rdma.wait()
                ocols = slice(base + cols.start, base + cols.stop)
                acc = comm[rs, :, cols] + out_ref[:, ocols].astype(jnp.int32)
                comm[rs, :, cols] = acc
                if last:
                    y = acc.astype(jnp.float32) * scale
                    out_ref[:, ocols] = y / (
                        1.0 + jnp.exp(-jnp.clip(y, -60.0, 60.0)))

    return pl.pallas_call(
        body,
        out_shape=jax.ShapeDtypeStruct((M_CHUNK, n), jnp.float32),
        in_specs=[
            pl.BlockSpec(memory_space=pltpu.VMEM),
            pl.BlockSpec(memory_space=pltpu.VMEM),
            pl.BlockSpec(memory_space=pltpu.SMEM),
            pl.BlockSpec(memory_space=pltpu.SMEM),
        ],
        out_specs=pl.BlockSpec(memory_space=pltpu.VMEM),
        scratch_shapes=[
            pltpu.VMEM((2, M_CHUNK, half), jnp.int32),
            pltpu.VMEM((2, M_CHUNK, half), jnp.int32),
            pltpu.SemaphoreType.DMA((2, 2)),
            pltpu.SemaphoreType.DMA((2, 2)),
            pltpu.SemaphoreType.DMA((N_DEV - 1, 2)),
            pltpu.SemaphoreType.DMA((N_DEV - 1, 2)),
        ],
        compiler_params=pltpu.CompilerParams(
            collective_id=0,
            vmem_limit_bytes=128 * 1024 * 1024,
        ),
    )(x, w_mat, scale_x, scale_w)


# device time: 679749 ns/iter; 1.9735x vs baseline; 1.0058x over previous
import jax
import jax.numpy as jnp
from jax import lax
from jax.experimental import pallas as pl
from jax.experimental.pallas import tpu as pltpu

N_DEV = 8
M_CHUNK = 512


def kernel(x, w_mat, scale_x, scale_w):
    m_total, k_per = x.shape
    _, n = w_mat.shape
    half = n // 2

    def body(x_ref, w_ref, sx_ref, sw_ref, out_ref, comm_r, comm_l,
             send_sems_r, send_sems_l, recv_sems_r, recv_sems_l):
        d = lax.axis_index("i")
        left = lax.rem(d - 1 + N_DEV, N_DEV)
        right = lax.rem(d + 1, N_DEV)

        barrier_sem = pltpu.get_barrier_semaphore()
        for nbr in (left, right):
            pl.semaphore_signal(
                barrier_sem, inc=1,
                device_id=(nbr,), device_id_type=pl.DeviceIdType.MESH,
            )
        pl.semaphore_wait(barrier_sem, 2)

        def partial(c, lo):
            xc = x_ref[pl.ds(c * M_CHUNK, M_CHUNK), :]
            wc = w_ref[:, lo:lo + half]
            return lax.dot_general(
                xc, wc,
                (((1,), (0,)), ((), ())),
                preferred_element_type=jnp.int32,
            )

        comm_r[0, :, :] = partial(lax.rem(d - 1 + N_DEV, N_DEV), 0)
        comm_l[0, :, :] = partial(lax.rem(d + 1, N_DEV), half)

        scale = sx_ref[0] * sw_ref[0]
        n_sub = 4
        sub = half // n_sub

        for s in range(N_DEV - 1):
            ss = s % 2
            rs = (s + 1) % 2
            descs = []
            for comm, ssems, rsems, tgt, base in (
                (comm_r, send_sems_r, recv_sems_r, right, 0),
                (comm_l, send_sems_l, recv_sems_l, left, half),
            ):
                for t in range(n_sub):
                    cols = slice(t * sub, (t + 1) * sub)
                    rdma = pltpu.make_async_remote_copy(
                        src_ref=comm.at[ss, :, cols],
                        dst_ref=comm.at[rs, :, cols],
                        send_sem=ssems.at[ss, t],
                        recv_sem=rsems.at[s, t],
                        device_id=(tgt,),
                        device_id_type=pl.DeviceIdType.MESH,
                    )
                    rdma.start()
                    descs.append((rdma, comm, cols, base))
            c_r = lax.rem(d - 2 - s + 2 * N_DEV, N_DEV)
            c_l = lax.rem(d + 2 + s, N_DEV)
            out_ref[:, 0:half] = partial(c_r, 0).astype(jnp.float32)
            out_ref[:, half:n] = partial(c_l, half).astype(jnp.float32)
            last = s == N_DEV - 2
            order = [descs[dir_ * n_sub + t]
                     for t in range(n_sub) for dir_ in range(2)]
            for rdma, comm, cols, base in order:
                rdma.wait()
                ocols = slice(base + cols.start, base + cols.stop)
                acc = comm[rs, :, cols] + out_ref[:, ocols].astype(jnp.int32)
                comm[rs, :, cols] = acc
                if last:
                    y = acc.astype(jnp.float32) * scale
                    out_ref[:, ocols] = y / (
                        1.0 + jnp.exp(-jnp.clip(y, -60.0, 60.0)))

    return pl.pallas_call(
        body,
        out_shape=jax.ShapeDtypeStruct((M_CHUNK, n), jnp.float32),
        in_specs=[
            pl.BlockSpec(memory_space=pltpu.VMEM),
            pl.BlockSpec(memory_space=pltpu.VMEM),
            pl.BlockSpec(memory_space=pltpu.SMEM),
            pl.BlockSpec(memory_space=pltpu.SMEM),
        ],
        out_specs=pl.BlockSpec(memory_space=pltpu.VMEM),
        scratch_shapes=[
            pltpu.VMEM((2, M_CHUNK, half), jnp.int32),
            pltpu.VMEM((2, M_CHUNK, half), jnp.int32),
            pltpu.SemaphoreType.DMA((2, 4)),
            pltpu.SemaphoreType.DMA((2, 4)),
            pltpu.SemaphoreType.DMA((N_DEV - 1, 4)),
            pltpu.SemaphoreType.DMA((N_DEV - 1, 4)),
        ],
        compiler_params=pltpu.CompilerParams(
            collective_id=0,
            vmem_limit_bytes=128 * 1024 * 1024,
        ),
    )(x, w_mat, scale_x, scale_w)


# device time: 664047 ns/iter; 2.0201x vs baseline; 1.0236x over previous
import jax
import jax.numpy as jnp
from jax import lax
from jax.experimental import pallas as pl
from jax.experimental.pallas import tpu as pltpu

N_DEV = 8
M_CHUNK = 512


def kernel(x, w_mat, scale_x, scale_w):
    m_total, k_per = x.shape
    _, n = w_mat.shape
    half = n // 2

    def body(x_ref, w_ref, sx_ref, sw_ref, out_ref, comm_r, comm_l,
             send_sems_r, send_sems_l, recv_sems_r, recv_sems_l):
        d = lax.axis_index("i")
        left = lax.rem(d - 1 + N_DEV, N_DEV)
        right = lax.rem(d + 1, N_DEV)

        barrier_sem = pltpu.get_barrier_semaphore()
        for nbr in (left, right):
            pl.semaphore_signal(
                barrier_sem, inc=1,
                device_id=(nbr,), device_id_type=pl.DeviceIdType.MESH,
            )
        pl.semaphore_wait(barrier_sem, 2)

        def partial(c, lo):
            xc = x_ref[pl.ds(c * M_CHUNK, M_CHUNK), :]
            wc = w_ref[:, lo:lo + half]
            return lax.dot_general(
                xc, wc,
                (((1,), (0,)), ((), ())),
                preferred_element_type=jnp.int32,
            )

        comm_r[0, :, :] = partial(lax.rem(d - 1 + N_DEV, N_DEV), 0)
        comm_l[0, :, :] = partial(lax.rem(d + 1, N_DEV), half)

        scale = sx_ref[0] * sw_ref[0]
        n_sub = 4
        sub = half // n_sub

        dirs = (
            (comm_r, send_sems_r, recv_sems_r, right, 0),
            (comm_l, send_sems_l, recv_sems_l, left, half),
        )

        def mk(s_, dir_, t):
            comm, ssems, rsems, tgt, _ = dirs[dir_]
            cols = slice(t * sub, (t + 1) * sub)
            return pltpu.make_async_remote_copy(
                src_ref=comm.at[s_ % 2, :, cols],
                dst_ref=comm.at[(s_ + 1) % 2, :, cols],
                send_sem=ssems.at[s_ % 2, t],
                recv_sem=rsems.at[s_, t],
                device_id=(tgt,),
                device_id_type=pl.DeviceIdType.MESH,
            )

        descs = {}
        for t in range(n_sub):
            for dir_ in range(2):
                r = mk(0, dir_, t)
                r.start()
                descs[(0, dir_, t)] = r

        for s in range(N_DEV - 1):
            rs = (s + 1) % 2
            c_r = lax.rem(d - 2 - s + 2 * N_DEV, N_DEV)
            c_l = lax.rem(d + 2 + s, N_DEV)
            out_ref[:, 0:half] = partial(c_r, 0).astype(jnp.float32)
            out_ref[:, half:n] = partial(c_l, half).astype(jnp.float32)
            last = s == N_DEV - 2
            for t in range(n_sub):
                for dir_ in range(2):
                    comm, _, _, _, base = dirs[dir_]
                    cols = slice(t * sub, (t + 1) * sub)
                    descs[(s, dir_, t)].wait_recv()
                    ocols = slice(base + cols.start, base + cols.stop)
                    acc = comm[rs, :, cols] + out_ref[:, ocols].astype(
                        jnp.int32)
                    comm[rs, :, cols] = acc
                    if last:
                        y = acc.astype(jnp.float32) * scale
                        out_ref[:, ocols] = y / (
                            1.0 + jnp.exp(-jnp.clip(y, -60.0, 60.0)))
                    else:
                        if s >= 1:
                            descs[(s - 1, dir_, t)].wait_send()
                        nxt = mk(s + 1, dir_, t)
                        nxt.start()
                        descs[(s + 1, dir_, t)] = nxt

        for t in range(n_sub):
            for dir_ in range(2):
                descs[(N_DEV - 3, dir_, t)].wait_send()
                descs[(N_DEV - 2, dir_, t)].wait_send()

    return pl.pallas_call(
        body,
        out_shape=jax.ShapeDtypeStruct((M_CHUNK, n), jnp.float32),
        in_specs=[
            pl.BlockSpec(memory_space=pltpu.VMEM),
            pl.BlockSpec(memory_space=pltpu.VMEM),
            pl.BlockSpec(memory_space=pltpu.SMEM),
            pl.BlockSpec(memory_space=pltpu.SMEM),
        ],
        out_specs=pl.BlockSpec(memory_space=pltpu.VMEM),
        scratch_shapes=[
            pltpu.VMEM((2, M_CHUNK, half), jnp.int32),
            pltpu.VMEM((2, M_CHUNK, half), jnp.int32),
            pltpu.SemaphoreType.DMA((2, 4)),
            pltpu.SemaphoreType.DMA((2, 4)),
            pltpu.SemaphoreType.DMA((N_DEV - 1, 4)),
            pltpu.SemaphoreType.DMA((N_DEV - 1, 4)),
        ],
        compiler_params=pltpu.CompilerParams(
            collective_id=0,
            vmem_limit_bytes=128 * 1024 * 1024,
        ),
    )(x, w_mat, scale_x, scale_w)


# device time: 659192 ns/iter; 2.0350x vs baseline; 1.0074x over previous
import jax
import jax.numpy as jnp
from jax import lax
from jax.experimental import pallas as pl
from jax.experimental.pallas import tpu as pltpu

N_DEV = 8
M_CHUNK = 512


def kernel(x, w_mat, scale_x, scale_w):
    m_total, k_per = x.shape
    _, n = w_mat.shape
    half = n // 2

    def body(x_ref, w_ref, sx_ref, sw_ref, out_ref, comm_r, comm_l,
             send_sems_r, send_sems_l, recv_sems_r, recv_sems_l):
        d = lax.axis_index("i")
        left = lax.rem(d - 1 + N_DEV, N_DEV)
        right = lax.rem(d + 1, N_DEV)

        barrier_sem = pltpu.get_barrier_semaphore()
        for nbr in (left, right):
            pl.semaphore_signal(
                barrier_sem, inc=1,
                device_id=(nbr,), device_id_type=pl.DeviceIdType.MESH,
            )
        pl.semaphore_wait(barrier_sem, 2)

        def partial(c, lo, width=None):
            width = half if width is None else width
            xc = x_ref[pl.ds(c * M_CHUNK, M_CHUNK), :]
            wc = w_ref[:, lo:lo + width]
            return lax.dot_general(
                xc, wc,
                (((1,), (0,)), ((), ())),
                preferred_element_type=jnp.int32,
            )

        scale = sx_ref[0] * sw_ref[0]
        n_sub = 8
        sub = half // n_sub

        dirs = (
            (comm_r, send_sems_r, recv_sems_r, right, 0),
            (comm_l, send_sems_l, recv_sems_l, left, half),
        )

        def mk(s_, dir_, t):
            comm, ssems, rsems, tgt, _ = dirs[dir_]
            cols = slice(t * sub, (t + 1) * sub)
            return pltpu.make_async_remote_copy(
                src_ref=comm.at[s_ % 2, :, cols],
                dst_ref=comm.at[(s_ + 1) % 2, :, cols],
                send_sem=ssems.at[s_ % 2, t],
                recv_sem=rsems.at[s_, t],
                device_id=(tgt,),
                device_id_type=pl.DeviceIdType.MESH,
            )

        seed_c = (lax.rem(d - 1 + N_DEV, N_DEV), lax.rem(d + 1, N_DEV))
        descs = {}
        for t in range(n_sub):
            for dir_ in range(2):
                comm, _, _, _, base = dirs[dir_]
                cols = slice(t * sub, (t + 1) * sub)
                comm[0, :, cols] = partial(seed_c[dir_], base + cols.start,
                                           sub)
                r = mk(0, dir_, t)
                r.start()
                descs[(0, dir_, t)] = r

        for s in range(N_DEV - 1):
            rs = (s + 1) % 2
            c_r = lax.rem(d - 2 - s + 2 * N_DEV, N_DEV)
            c_l = lax.rem(d + 2 + s, N_DEV)
            out_ref[:, 0:half] = partial(c_r, 0).astype(jnp.float32)
            out_ref[:, half:n] = partial(c_l, half).astype(jnp.float32)
            last = s == N_DEV - 2
            for t in range(n_sub):
                for dir_ in range(2):
                    comm, _, _, _, base = dirs[dir_]
                    cols = slice(t * sub, (t + 1) * sub)
                    descs[(s, dir_, t)].wait_recv()
                    ocols = slice(base + cols.start, base + cols.stop)
                    acc = comm[rs, :, cols] + out_ref[:, ocols].astype(
                        jnp.int32)
                    comm[rs, :, cols] = acc
                    if last:
                        y = acc.astype(jnp.float32) * scale
                        out_ref[:, ocols] = y / (
                            1.0 + jnp.exp(-jnp.clip(y, -60.0, 60.0)))
                    else:
                        if s >= 1:
                            descs[(s - 1, dir_, t)].wait_send()
                        nxt = mk(s + 1, dir_, t)
                        nxt.start()
                        descs[(s + 1, dir_, t)] = nxt

        for t in range(n_sub):
            for dir_ in range(2):
                descs[(N_DEV - 3, dir_, t)].wait_send()
                descs[(N_DEV - 2, dir_, t)].wait_send()

    return pl.pallas_call(
        body,
        out_shape=jax.ShapeDtypeStruct((M_CHUNK, n), jnp.float32),
        in_specs=[
            pl.BlockSpec(memory_space=pltpu.VMEM),
            pl.BlockSpec(memory_space=pltpu.VMEM),
            pl.BlockSpec(memory_space=pltpu.SMEM),
            pl.BlockSpec(memory_space=pltpu.SMEM),
        ],
        out_specs=pl.BlockSpec(memory_space=pltpu.VMEM),
        scratch_shapes=[
            pltpu.VMEM((2, M_CHUNK, half), jnp.int32),
            pltpu.VMEM((2, M_CHUNK, half), jnp.int32),
            pltpu.SemaphoreType.DMA((2, 8)),
            pltpu.SemaphoreType.DMA((2, 8)),
            pltpu.SemaphoreType.DMA((N_DEV - 1, 8)),
            pltpu.SemaphoreType.DMA((N_DEV - 1, 8)),
        ],
        compiler_params=pltpu.CompilerParams(
            collective_id=0,
            vmem_limit_bytes=128 * 1024 * 1024,
        ),
    )(x, w_mat, scale_x, scale_w)
